# Initial kernel scaffold; baseline (speedup 1.0000x reference)
import jax
import jax.numpy as jnp
from jax.experimental import pallas as pl


def kernel(x, b_pre, W_enc, b_enc, W_dec):
    raise NotImplementedError("write your pallas kernel here")


# speedup vs baseline: 26.1768x; 26.1768x over previous
import functools

import jax
import jax.numpy as jnp
from jax.experimental import pallas as pl
from jax.experimental.pallas import tpu as pltpu

B = 16384
D = 64
L = 1024
K = 8
BM = 256


def _fused_body(x_ref, b_pre_ref, W_enc_ref, b_enc_ref, W_dec_ref,
                acts_ref, recon_ref):
    x = x_ref[...]
    b_pre = b_pre_ref[...]
    W_enc = W_enc_ref[...]
    b_enc = b_enc_ref[...]
    W_dec = W_dec_ref[...]

    pre = jnp.dot(x - b_pre, W_enc,
                  preferred_element_type=jnp.float32) + b_enc

    iota = jax.lax.broadcasted_iota(jnp.int32, (BM, L), 1)
    neg_inf = jnp.float32(-jnp.inf)
    big = jnp.int32(L)

    work = pre
    mask = jnp.zeros((BM, L), dtype=jnp.bool_)
    for _ in range(K):
        m = jnp.max(work, axis=1, keepdims=True)
        cand = work == m
        first = jnp.min(jnp.where(cand, iota, big), axis=1,
                        keepdims=True)
        sel = iota == first
        mask = jnp.logical_or(mask, sel)
        work = jnp.where(sel, neg_inf, work)

    acts = jnp.where(jnp.logical_and(mask, pre > 0), pre,
                     jnp.float32(0.0))
    acts_ref[...] = acts
    recon_ref[...] = jnp.dot(acts, W_dec,
                             preferred_element_type=jnp.float32) + b_pre


@jax.jit
def kernel(x, b_pre, W_enc, b_enc, W_dec):
    grid = (B // BM,)
    acts, recon = pl.pallas_call(
        _fused_body,
        grid=grid,
        in_specs=[
            pl.BlockSpec((BM, D), lambda i: (i, 0)),
            pl.BlockSpec((1, D), lambda i: (0, 0)),
            pl.BlockSpec((D, L), lambda i: (0, 0)),
            pl.BlockSpec((1, L), lambda i: (0, 0)),
            pl.BlockSpec((L, D), lambda i: (0, 0)),
        ],
        out_specs=[
            pl.BlockSpec((BM, L), lambda i: (i, 0)),
            pl.BlockSpec((BM, D), lambda i: (i, 0)),
        ],
        out_shape=[
            jax.ShapeDtypeStruct((B, L), jnp.float32),
            jax.ShapeDtypeStruct((B, D), jnp.float32),
        ],
        compiler_params=pltpu.CompilerParams(
            dimension_semantics=("arbitrary",),
        ),
    )(x, b_pre.reshape(1, D), W_enc, b_enc.reshape(1, L), W_dec)
    return acts, recon


# speedup vs baseline: 64.4110x; 2.4606x over previous
import functools

import jax
import jax.numpy as jnp
from jax.experimental import pallas as pl
from jax.experimental.pallas import tpu as pltpu

B = 16384
D = 64
L = 1024
K = 8
BM = 256


def _fused_body(x_ref, b_pre_ref, W_enc_ref, b_enc_ref, W_dec_ref,
                acts_ref, recon_ref):
    x = x_ref[...]
    b_pre = b_pre_ref[...]
    W_enc = W_enc_ref[...]
    b_enc = b_enc_ref[...]
    W_dec = W_dec_ref[...]

    pre = jnp.dot(x - b_pre, W_enc,
                  preferred_element_type=jnp.float32) + b_enc

    neg_inf = jnp.float32(-jnp.inf)

    work = pre
    for r in range(K):
        t = jnp.max(work, axis=1, keepdims=True)
        if r < K - 1:
            work = jnp.where(work == t, neg_inf, work)

    keep = jnp.logical_and(pre >= t, pre > 0)
    acts = jnp.where(keep, pre, jnp.float32(0.0))
    acts_ref[...] = acts
    recon_ref[...] = jnp.dot(acts, W_dec,
                             preferred_element_type=jnp.float32) + b_pre


@jax.jit
def kernel(x, b_pre, W_enc, b_enc, W_dec):
    grid = (B // BM,)
    acts, recon = pl.pallas_call(
        _fused_body,
        grid=grid,
        in_specs=[
            pl.BlockSpec((BM, D), lambda i: (i, 0)),
            pl.BlockSpec((1, D), lambda i: (0, 0)),
            pl.BlockSpec((D, L), lambda i: (0, 0)),
            pl.BlockSpec((1, L), lambda i: (0, 0)),
            pl.BlockSpec((L, D), lambda i: (0, 0)),
        ],
        out_specs=[
            pl.BlockSpec((BM, L), lambda i: (i, 0)),
            pl.BlockSpec((BM, D), lambda i: (i, 0)),
        ],
        out_shape=[
            jax.ShapeDtypeStruct((B, L), jnp.float32),
            jax.ShapeDtypeStruct((B, D), jnp.float32),
        ],
        compiler_params=pltpu.CompilerParams(
            dimension_semantics=("arbitrary",),
        ),
    )(x, b_pre.reshape(1, D), W_enc, b_enc.reshape(1, L), W_dec)
    return acts, recon


# speedup vs baseline: 66.3626x; 1.0303x over previous
import functools

import jax
import jax.numpy as jnp
from jax.experimental import pallas as pl
from jax.experimental.pallas import tpu as pltpu

B = 16384
D = 64
L = 1024
K = 8
BM = 256


def _fused_body(x_ref, b_pre_ref, W_enc_ref, b_enc_ref, W_dec_ref,
                acts_ref, recon_ref):
    x = x_ref[...]
    b_pre = b_pre_ref[...]
    W_enc = W_enc_ref[...]
    b_enc = b_enc_ref[...]
    W_dec = W_dec_ref[...]

    pre = jnp.dot(x - b_pre, W_enc,
                  preferred_element_type=jnp.float32) + b_enc

    NC = L // 128
    S = [pre[:, c * 128:(c + 1) * 128] for c in range(NC)]
    net = [(0, 1), (2, 3), (4, 5), (6, 7),
           (0, 2), (1, 3), (4, 6), (5, 7),
           (1, 2), (5, 6), (0, 4), (3, 7),
           (1, 5), (2, 6),
           (1, 4), (3, 6),
           (2, 4), (3, 5),
           (3, 4)]
    for i, j in net:
        hi = jnp.maximum(S[i], S[j])
        lo = jnp.minimum(S[i], S[j])
        S[i], S[j] = hi, lo

    for r in range(K - 1):
        t = jnp.max(S[0], axis=1, keepdims=True)
        pop = S[0] == t
        for i in range(K - 1 - r):
            S[i] = jnp.where(pop, S[i + 1], S[i])
    t = jnp.max(S[0], axis=1, keepdims=True)

    keep = jnp.logical_and(pre >= t, pre > 0)
    acts = jnp.where(keep, pre, jnp.float32(0.0))
    acts_ref[...] = acts
    recon_ref[...] = jnp.dot(acts, W_dec,
                             preferred_element_type=jnp.float32) + b_pre


@jax.jit
def kernel(x, b_pre, W_enc, b_enc, W_dec):
    grid = (B // BM,)
    acts, recon = pl.pallas_call(
        _fused_body,
        grid=grid,
        in_specs=[
            pl.BlockSpec((BM, D), lambda i: (i, 0)),
            pl.BlockSpec((1, D), lambda i: (0, 0)),
            pl.BlockSpec((D, L), lambda i: (0, 0)),
            pl.BlockSpec((1, L), lambda i: (0, 0)),
            pl.BlockSpec((L, D), lambda i: (0, 0)),
        ],
        out_specs=[
            pl.BlockSpec((BM, L), lambda i: (i, 0)),
            pl.BlockSpec((BM, D), lambda i: (i, 0)),
        ],
        out_shape=[
            jax.ShapeDtypeStruct((B, L), jnp.float32),
            jax.ShapeDtypeStruct((B, D), jnp.float32),
        ],
        compiler_params=pltpu.CompilerParams(
            dimension_semantics=("arbitrary",),
        ),
    )(x, b_pre.reshape(1, D), W_enc, b_enc.reshape(1, L), W_dec)
    return acts, recon
